# baseline (speedup 1.0000x reference)
import functools

import jax
import jax.numpy as jnp
from jax import lax
from jax.experimental import pallas as pl
from jax.experimental.pallas import tpu as pltpu
from jax.experimental.pallas import tpu_sc as plsc


def _sc_gather(table, idx, n, k):
    info = plsc.get_sparse_core_info()
    nw = info.num_cores * info.num_subcores
    b_per_w = n // nw

    mesh = plsc.VectorSubcoreMesh(core_axis_name="c", subcore_axis_name="s")

    @functools.partial(
        pl.kernel,
        mesh=mesh,
        compiler_params=pltpu.CompilerParams(use_tc_tiling_on_sc=False),
        out_type=jax.ShapeDtypeStruct((n, k), jnp.float32),
        scratch_types=[
            pltpu.VMEM((b_per_w,), jnp.int32),
            pltpu.VMEM((b_per_w, k), jnp.float32),
            pltpu.SemaphoreType.DMA,
        ],
    )
    def gather_kernel(table_hbm, idx_hbm, out_hbm, idx_v, rows_v, sem):
        wid = lax.axis_index("s") * info.num_cores + lax.axis_index("c")
        base = wid * b_per_w
        pltpu.sync_copy(idx_hbm.at[pl.ds(base, b_per_w)], idx_v)
        pltpu.async_copy(table_hbm.at[idx_v], rows_v, sem).wait()
        pltpu.sync_copy(rows_v, out_hbm.at[pl.ds(base, b_per_w)])

    return gather_kernel(table, idx)


def _tc_matmul(g128, r_big, n_lines, width):
    block_m = n_lines

    def mm_body(g_ref, r_ref, o_ref):
        o_ref[...] = jnp.dot(g_ref[...], r_ref[...],
                             preferred_element_type=jnp.float32
                             ).astype(o_ref.dtype)

    return pl.pallas_call(
        mm_body,
        grid=(n_lines // block_m,),
        in_specs=[
            pl.BlockSpec((block_m, 128), lambda i: (i, 0)),
            pl.BlockSpec((128, width), lambda i: (0, 0)),
        ],
        out_specs=pl.BlockSpec((block_m, width), lambda i: (i, 0)),
        out_shape=jax.ShapeDtypeStruct((n_lines, width), jnp.bfloat16),
    )(g128, r_big)


def kernel(x, L, R):
    b, h = x.shape
    v, k = L.shape
    d, _ = R.shape
    n = b * h
    pack = 128 // k

    idx = x.reshape(n).astype(jnp.int32)
    g = _sc_gather(L, idx, n, k)
    g128 = g.reshape(n // pack, 128)
    r_big = jnp.kron(jnp.eye(pack, dtype=jnp.float32), R.T)
    out512 = _tc_matmul(g128, r_big, n // pack, pack * d)
    return out512.reshape(b, h, d).astype(jnp.float32)


# speedup vs baseline: 1.0220x; 1.0220x over previous
import functools

import jax
import jax.numpy as jnp
from jax import lax
from jax.experimental import pallas as pl
from jax.experimental.pallas import tpu as pltpu
from jax.experimental.pallas import tpu_sc as plsc


def _sc_gather(table, idx, n, k):
    info = plsc.get_sparse_core_info()
    nw = info.num_cores * info.num_subcores
    b_per_w = n // nw

    mesh = plsc.VectorSubcoreMesh(core_axis_name="c", subcore_axis_name="s")

    @functools.partial(
        pl.kernel,
        mesh=mesh,
        compiler_params=pltpu.CompilerParams(use_tc_tiling_on_sc=False),
        out_type=jax.ShapeDtypeStruct((n, k), jnp.float32),
        scratch_types=[
            pltpu.VMEM((b_per_w,), jnp.int32),
            pltpu.VMEM((b_per_w, k), jnp.float32),
            pltpu.SemaphoreType.DMA,
        ],
    )
    def gather_kernel(table_hbm, idx_hbm, out_hbm, idx_v, rows_v, sem):
        wid = lax.axis_index("s") * info.num_cores + lax.axis_index("c")
        base = wid * b_per_w
        pltpu.sync_copy(idx_hbm.at[pl.ds(base, b_per_w)], idx_v)
        pltpu.async_copy(table_hbm.at[idx_v], rows_v, sem).wait()
        pltpu.sync_copy(rows_v, out_hbm.at[pl.ds(base, b_per_w)])

    return gather_kernel(table, idx)


def _tc_matmul(g128, r_big, n_lines, width):
    block_m = n_lines // 2

    def mm_body(g_ref, r_ref, o_ref):
        o_ref[...] = jnp.dot(g_ref[...], r_ref[...],
                             preferred_element_type=jnp.float32
                             ).astype(o_ref.dtype)

    return pl.pallas_call(
        mm_body,
        grid=(n_lines // block_m,),
        in_specs=[
            pl.BlockSpec((block_m, 128), lambda i: (i, 0)),
            pl.BlockSpec((128, width), lambda i: (0, 0)),
        ],
        out_specs=pl.BlockSpec((block_m, width), lambda i: (i, 0)),
        out_shape=jax.ShapeDtypeStruct((n_lines, width), jnp.bfloat16),
    )(g128, r_big)


def kernel(x, L, R):
    b, h = x.shape
    v, k = L.shape
    d, _ = R.shape
    n = b * h
    pack = 128 // k

    idx = x.reshape(n).astype(jnp.int32)
    g = _sc_gather(L, idx, n, k)
    g128 = g.reshape(n // pack, 128)
    r_big = jnp.kron(jnp.eye(pack, dtype=jnp.float32), R.T)
    out512 = _tc_matmul(g128, r_big, n // pack, pack * d)
    return out512.reshape(b, h, d).astype(jnp.float32)
